# Initial kernel scaffold; baseline (speedup 1.0000x reference)
import jax
import jax.numpy as jnp
from jax.experimental import pallas as pl


def kernel(probs):
    raise NotImplementedError("write your pallas kernel here")


# speedup vs baseline: 14.8885x; 14.8885x over previous
import functools

import jax
import jax.numpy as jnp
from jax import lax
from jax.experimental import pallas as pl
from jax.experimental.pallas import tpu as pltpu
from jax.experimental.pallas import tpu_sc as plsc

_THRESH = 0.5
_B = 64
_N = 32768
_L = 16
_CHUNKS = _N // _L


def _tec_body(probs_hbm, out_hbm, pv, ov):
    cid = lax.axis_index("c")
    sid = lax.axis_index("s")
    wid = sid * 2 + cid
    neg1 = jnp.full((_L,), -1, jnp.int32)
    lane = lax.iota(jnp.int32, _L)

    def do_row(r):
        pltpu.sync_copy(probs_hbm.at[r], pv)

        def ibody(j, carry):
            ov[pl.ds(j * _L, _L)] = neg1
            return carry

        lax.fori_loop(0, _CHUNKS, ibody, 0, unroll=8)

        def cbody(i, base):
            v = pv[pl.ds(i * _L, _L)]
            m = v >= _THRESH
            cs = plsc.cumsum(m.astype(jnp.int32))
            pos = cs + base - 1
            idxs = lane + i * _L
            plsc.store_scatter(ov, [pos], idxs, mask=m)
            cnt = plsc.all_reduce_population_count(m)
            return base + cnt

        lax.fori_loop(0, _CHUNKS, cbody, jnp.zeros((_L,), jnp.int32),
                      unroll=4)
        pltpu.sync_copy(ov, out_hbm.at[r])

    do_row(wid * 2)
    do_row(wid * 2 + 1)


_fn_cache = []


def _get_fn():
    if not _fn_cache:
        mesh = plsc.VectorSubcoreMesh(core_axis_name="c",
                                      subcore_axis_name="s")
        fn = functools.partial(
            pl.kernel,
            out_type=jax.ShapeDtypeStruct((_B, _N), jnp.int32),
            mesh=mesh,
            scratch_types=[
                pltpu.VMEM((_N,), jnp.float32),
                pltpu.VMEM((_N,), jnp.int32),
            ],
            compiler_params=pltpu.CompilerParams(needs_layout_passes=False),
        )(_tec_body)
        _fn_cache.append(fn)
    return _fn_cache[0]


def kernel(probs):
    return _get_fn()(probs)


# speedup vs baseline: 16.3872x; 1.1007x over previous
import functools

import jax
import jax.numpy as jnp
from jax import lax
from jax.experimental import pallas as pl
from jax.experimental.pallas import tpu as pltpu
from jax.experimental.pallas import tpu_sc as plsc

_THRESH = 0.5
_B = 64
_N = 32768
_L = 16
_CHUNKS = _N // _L


def _tec_body(probs_hbm, out_hbm, pv, ov):
    cid = lax.axis_index("c")
    sid = lax.axis_index("s")
    wid = sid * 2 + cid
    neg1 = jnp.full((_L,), -1, jnp.int32)
    ones = jnp.full((_L,), 1, jnp.int32)
    sixteen = jnp.full((_L,), _L, jnp.int32)
    lane = lax.iota(jnp.int32, _L)

    def do_row(r):
        pltpu.sync_copy(probs_hbm.at[r], pv)

        def cbody(i, carry):
            basem1, idxs = carry
            v = pv[pl.ds(i * _L, _L)]
            m = v >= _THRESH
            ov[pl.ds(i * _L, _L)] = neg1
            cs = plsc.cumsum(ones, mask=m)
            pos = cs + basem1
            plsc.store_scatter(ov, [pos], idxs, mask=m)
            cnt = plsc.all_reduce_population_count(m)
            return (basem1 + cnt, idxs + sixteen)

        lax.fori_loop(0, _CHUNKS, cbody,
                      (jnp.full((_L,), -1, jnp.int32), lane), unroll=8)
        pltpu.sync_copy(ov, out_hbm.at[r])

    do_row(wid * 2)
    do_row(wid * 2 + 1)


_fn_cache = []


def _get_fn():
    if not _fn_cache:
        mesh = plsc.VectorSubcoreMesh(core_axis_name="c",
                                      subcore_axis_name="s")
        fn = functools.partial(
            pl.kernel,
            out_type=jax.ShapeDtypeStruct((_B, _N), jnp.int32),
            mesh=mesh,
            scratch_types=[
                pltpu.VMEM((_N,), jnp.float32),
                pltpu.VMEM((_N,), jnp.int32),
            ],
            compiler_params=pltpu.CompilerParams(needs_layout_passes=False),
        )(_tec_body)
        _fn_cache.append(fn)
    return _fn_cache[0]


def kernel(probs):
    return _get_fn()(probs)


# speedup vs baseline: 26.2567x; 1.6023x over previous
import functools

import jax
import jax.numpy as jnp
from jax import lax
from jax.experimental import pallas as pl
from jax.experimental.pallas import tpu as pltpu
from jax.experimental.pallas import tpu_sc as plsc

_THRESH = 0.5
_B = 64
_N = 32768
_L = 16
_CHUNKS = _N // _L


def _tec_body(probs_hbm, out_hbm, pv, ov):
    cid = lax.axis_index("c")
    sid = lax.axis_index("s")
    wid = sid * 2 + cid
    neg1 = jnp.full((_L,), -1, jnp.int32)
    zeros = jnp.zeros((_L,), jnp.int32)
    ones = jnp.full((_L,), 1, jnp.int32)
    sixteen = jnp.full((_L,), _L, jnp.int32)
    lane = lax.iota(jnp.int32, _L)
    shift_idx = [jnp.maximum(lane - k, 0) for k in (1, 2, 4, 8)]
    shift_ok = [lane >= k for k in (1, 2, 4, 8)]

    def prefix_sum(s):
        for idxk, okk in zip(shift_idx, shift_ok):
            sh = jnp.take_along_axis(s, idxk, axis=0)
            s = s + jnp.where(okk, sh, zeros)
        return s

    _G = 8

    def do_row(r):
        pltpu.sync_copy(probs_hbm.at[r], pv)

        def cbody(it, carry):
            basem1, idxs = carry
            i0 = it * _G
            vs = [pv[pl.ds((i0 + g) * _L, _L)] for g in range(_G)]
            ms = [v >= _THRESH for v in vs]
            ss = [jnp.where(m, ones, zeros) for m in ms]
            for idxk, okk in zip(shift_idx, shift_ok):
                shs = [jnp.take_along_axis(s, idxk, axis=0) for s in ss]
                ss = [s + jnp.where(okk, sh, zeros)
                      for s, sh in zip(ss, shs)]
            cnts = [plsc.all_reduce_population_count(m) for m in ms]
            bases = [basem1]
            for g in range(_G):
                bases.append(bases[g] + cnts[g])
            vals = [idxs + jnp.full((_L,), g * _L, jnp.int32)
                    for g in range(_G)]
            for g in range(_G):
                ov[pl.ds((i0 + g) * _L, _L)] = neg1
                plsc.store_scatter(ov, [ss[g] + bases[g]], vals[g],
                                   mask=ms[g])
            return (bases[_G], idxs + jnp.full((_L,), _G * _L, jnp.int32))

        lax.fori_loop(0, _CHUNKS // _G, cbody,
                      (jnp.full((_L,), -1, jnp.int32), lane))
        pltpu.sync_copy(ov, out_hbm.at[r])

    do_row(wid * 2)
    do_row(wid * 2 + 1)


_fn_cache = []


def _get_fn():
    if not _fn_cache:
        mesh = plsc.VectorSubcoreMesh(core_axis_name="c",
                                      subcore_axis_name="s")
        fn = functools.partial(
            pl.kernel,
            out_type=jax.ShapeDtypeStruct((_B, _N), jnp.int32),
            mesh=mesh,
            scratch_types=[
                pltpu.VMEM((_N,), jnp.float32),
                pltpu.VMEM((_N,), jnp.int32),
            ],
            compiler_params=pltpu.CompilerParams(needs_layout_passes=False),
        )(_tec_body)
        _fn_cache.append(fn)
    return _fn_cache[0]


def kernel(probs):
    return _get_fn()(probs)


# speedup vs baseline: 30.6785x; 1.1684x over previous
import functools

import jax
import jax.numpy as jnp
from jax import lax
from jax.experimental import pallas as pl
from jax.experimental.pallas import tpu as pltpu
from jax.experimental.pallas import tpu_sc as plsc

_THRESH = 0.5
_B = 64
_N = 32768
_L = 16
_CHUNKS = _N // _L


def _tec_body(probs_hbm, out_hbm, pv, ov):
    cid = lax.axis_index("c")
    sid = lax.axis_index("s")
    wid = sid * 2 + cid
    neg1 = jnp.full((_L,), -1, jnp.int32)
    zeros = jnp.zeros((_L,), jnp.int32)
    lane = lax.iota(jnp.int32, _L)
    lane16 = lane + jnp.full((_L,), _L, jnp.int32)

    _G = 8

    def do_row(r):
        pltpu.sync_copy(probs_hbm.at[r], pv)

        def cbody(it, carry):
            base0, idxs = carry
            i0 = it * _G
            vs = [pv[pl.ds((i0 + g) * _L, _L)] for g in range(_G)]
            ms = [v >= _THRESH for v in vs]
            keys = [jnp.where(m, lane, lane16) for m in ms]
            vals = [jnp.where(m, idxs + jnp.full((_L,), g * _L, jnp.int32),
                              neg1) for g, m in enumerate(ms)]
            sorted_vals = [plsc.sort_key_val(k, v)[1]
                           for k, v in zip(keys, vals)]
            cnts = [plsc.all_reduce_population_count(m) for m in ms]
            bases = [base0]
            for g in range(_G):
                bases.append(bases[g] + cnts[g])
            for g in range(_G):
                ov[pl.ds((i0 + g) * _L, _L)] = neg1
                plsc.store_scatter(ov, [bases[g] + lane], sorted_vals[g])
            return (bases[_G], idxs + jnp.full((_L,), _G * _L, jnp.int32))

        lax.fori_loop(0, _CHUNKS // _G, cbody, (zeros, lane))
        pltpu.sync_copy(ov, out_hbm.at[r])

    do_row(wid * 2)
    do_row(wid * 2 + 1)


_fn_cache = []


def _get_fn():
    if not _fn_cache:
        mesh = plsc.VectorSubcoreMesh(core_axis_name="c",
                                      subcore_axis_name="s")
        fn = functools.partial(
            pl.kernel,
            out_type=jax.ShapeDtypeStruct((_B, _N), jnp.int32),
            mesh=mesh,
            scratch_types=[
                pltpu.VMEM((_N,), jnp.float32),
                pltpu.VMEM((_N,), jnp.int32),
            ],
            compiler_params=pltpu.CompilerParams(needs_layout_passes=False),
        )(_tec_body)
        _fn_cache.append(fn)
    return _fn_cache[0]


def kernel(probs):
    return _get_fn()(probs)
